# baseline (speedup 1.0000x reference)
import functools

import jax
import jax.numpy as jnp
from jax import lax
from jax.experimental import pallas as pl
from jax.experimental.pallas import tpu as pltpu
from jax.experimental.pallas import tpu_sc as plsc

_GH, _GW = 64, 32
_NC, _NS = 2, 16
_NW = _NC * _NS


def _sdinv(vv):
    ii = jnp.right_shift(vv, 5)
    jj = jnp.bitwise_and(vv, _GW - 1)
    deg = ((ii > 0).astype(jnp.int32) + (ii < _GH - 1).astype(jnp.int32)
           + (jj > 0).astype(jnp.int32) + (jj < _GW - 1).astype(jnp.int32))
    return jnp.where(deg == 2, jnp.float32(0.70710678),
                     jnp.where(deg == 3, jnp.float32(0.57735027),
                               jnp.float32(0.5)))


def _sc_stencil_body(rows, x_hbm, out_hbm, xb, ob):
    wid = lax.axis_index("s") * _NC + lax.axis_index("c")
    per_w = rows // _NW
    chunk = 128
    halo = _GW
    buf_rows = chunk + 2 * halo
    v_mask = _GH * _GW - 1

    for cidx in range(per_w // chunk):
        chunk_start = wid * per_w + cidx * chunk
        s = pl.multiple_of(
            jnp.clip(chunk_start - halo, 0, rows - buf_rows), halo)
        off = chunk_start - s
        pltpu.sync_copy(x_hbm.at[pl.ds(s, buf_rows)], xb)

        def row_body(l, carry):
            r = chunk_start + l
            v = jnp.bitwise_and(r, v_mask)
            ii = jnp.right_shift(v, 5)
            jj = jnp.bitwise_and(v, _GW - 1)
            dv = _sdinv(v)
            cm32 = (ii > 0).astype(jnp.float32) * dv * _sdinv(v - _GW)
            cp32 = (ii < _GH - 1).astype(jnp.float32) * dv * _sdinv(v + _GW)
            cm1 = (jj > 0).astype(jnp.float32) * dv * _sdinv(v - 1)
            cp1 = (jj < _GW - 1).astype(jnp.float32) * dv * _sdinv(v + 1)
            lc = l + off
            lm32 = jnp.maximum(lc - halo, 0)
            lp32 = jnp.minimum(lc + halo, buf_rows - 1)
            lm1 = jnp.maximum(lc - 1, 0)
            lp1 = jnp.minimum(lc + 1, buf_rows - 1)
            for k in range(8):
                cs = pl.ds(k * 16, 16)
                ob[l, cs] = (cm32 * xb[lm32, cs] + cm1 * xb[lm1, cs]
                             + cp1 * xb[lp1, cs] + cp32 * xb[lp32, cs])
            return carry

        lax.fori_loop(0, chunk, row_body, 0)
        pltpu.sync_copy(ob, out_hbm.at[pl.ds(chunk_start, chunk)])


def _sc_stencil(x):
    rows, d = x.shape
    body = functools.partial(_sc_stencil_body, rows)
    fn = pl.kernel(
        body,
        out_type=jax.ShapeDtypeStruct((rows, d), jnp.float32),
        mesh=plsc.VectorSubcoreMesh(core_axis_name="c", subcore_axis_name="s"),
        scratch_types=[
            pltpu.VMEM((128 + 2 * _GW, d), jnp.float32),
            pltpu.VMEM((128, d), jnp.float32),
        ],
    )
    return fn(x)


def _tc_layer_body(x_ref, xnei_ref, ws_ref, wn_ref, g_ref, b_ref, out_ref):
    dims = (((1,), (1,)), ((), ()))
    y = (lax.dot_general(x_ref[...].astype(jnp.bfloat16),
                         ws_ref[...].astype(jnp.bfloat16), dims,
                         preferred_element_type=jnp.float32)
         + lax.dot_general(xnei_ref[...].astype(jnp.bfloat16),
                           wn_ref[...].astype(jnp.bfloat16), dims,
                           preferred_element_type=jnp.float32))
    cnt = jnp.float32(y.shape[0])
    mu = jnp.sum(y, axis=0, keepdims=True) / cnt
    sq = jnp.sum(y * y, axis=0, keepdims=True) / cnt
    var = sq - mu * mu
    scale = lax.rsqrt(var + 1e-5) * g_ref[...]
    offv = b_ref[...] - mu * scale
    out_ref[...] = jnp.maximum(y * scale + offv, 0.0)


def _tc_layer(x, xnei, ws, wn, g, b):
    rows, d = x.shape
    return pl.pallas_call(
        _tc_layer_body,
        out_shape=jax.ShapeDtypeStruct((rows, d), jnp.float32),
    )(x, xnei, ws, wn, g.reshape(1, d), b.reshape(1, d))


def kernel(H, A_norm, Ws0, Wn0, g0, b0, Ws1, Wn1, g1, b1):
    n, v, d = H.shape
    x = H.reshape(n * v, d)
    for ws, wn, g, b in ((Ws0, Wn0, g0, b0), (Ws1, Wn1, g1, b1)):
        xnei = _sc_stencil(x)
        x = _tc_layer(x, xnei, ws, wn, g, b)
    return x.reshape(n, v, d)


# speedup vs baseline: 6.5923x; 6.5923x over previous
import functools

import jax
import jax.numpy as jnp
from jax.experimental import pallas as pl

_GH, _GW = 64, 32


def _gnn_body(n, v, d, h_ref,
              ws0_ref, wn0_ref, g0_ref, b0_ref,
              ws1_ref, wn1_ref, g1_ref, b1_ref, out_ref):
    gh, gw = _GH, _GW
    vi = jax.lax.broadcasted_iota(jnp.int32, (v, d), 0)
    gi = vi // gw
    gj = vi % gw
    deg = ((gi > 0).astype(jnp.float32) + (gi < gh - 1).astype(jnp.float32)
           + (gj > 0).astype(jnp.float32) + (gj < gw - 1).astype(jnp.float32))
    dinv = jax.lax.rsqrt(deg).astype(jnp.bfloat16)
    dinv4 = dinv.reshape(1, gh, gw, d)
    ones_row = jnp.ones((1, n * v), dtype=jnp.bfloat16)

    X = h_ref[...].astype(jnp.bfloat16)
    zi = jnp.zeros((n, 1, gw, d), dtype=jnp.bfloat16)
    zj = jnp.zeros((n, gh, 1, d), dtype=jnp.bfloat16)

    layers = ((ws0_ref, wn0_ref, g0_ref, b0_ref),
              (ws1_ref, wn1_ref, g1_ref, b1_ref))
    for ws_ref, wn_ref, g_ref, b_ref in layers:
        xs = X.reshape(n, gh, gw, d) * dinv4
        u = (jnp.concatenate([zi, xs[:, :-1]], axis=1)
             + jnp.concatenate([xs[:, 1:], zi], axis=1)
             + jnp.concatenate([zj, xs[:, :, :-1]], axis=2)
             + jnp.concatenate([xs[:, :, 1:], zj], axis=2))
        xnei = (u * dinv4).reshape(n * v, d)

        dims = (((1,), (1,)), ((), ()))
        y = (jax.lax.dot_general(X.reshape(n * v, d),
                                 ws_ref[...].astype(jnp.bfloat16), dims,
                                 preferred_element_type=jnp.float32)
             + jax.lax.dot_general(xnei, wn_ref[...].astype(jnp.bfloat16),
                                   dims, preferred_element_type=jnp.float32))

        yb = y.astype(jnp.bfloat16)
        cnt = jnp.float32(n * v)
        dims_nn = (((1,), (0,)), ((), ()))
        mu = jax.lax.dot_general(ones_row, yb, dims_nn,
                                 preferred_element_type=jnp.float32) / cnt
        sq = jax.lax.dot_general(ones_row, yb * yb, dims_nn,
                                 preferred_element_type=jnp.float32) / cnt
        var = sq - mu * mu
        scale = jax.lax.rsqrt(var + 1e-5) * g_ref[...]
        off = b_ref[...] - mu * scale
        xf = jnp.maximum(y * scale + off, 0.0)
        X = xf.astype(jnp.bfloat16).reshape(n, v, d)

    out_ref[...] = xf.reshape(n, v, d)


def kernel(H, A_norm, Ws0, Wn0, g0, b0, Ws1, Wn1, g1, b1):
    n, v, d = H.shape
    body = functools.partial(_gnn_body, n, v, d)
    return pl.pallas_call(
        body,
        out_shape=jax.ShapeDtypeStruct((n, v, d), jnp.float32),
    )(H, Ws0, Wn0, g0.reshape(1, d), b0.reshape(1, d),
      Ws1, Wn1, g1.reshape(1, d), b1.reshape(1, d))


# speedup vs baseline: 7.5682x; 1.1480x over previous
import functools

import jax
import jax.numpy as jnp
from jax.experimental import pallas as pl
from jax.experimental.pallas import tpu as pltpu

_GH, _GW = 64, 32


def _gnn_body(n, v, d, h_hbm,
              ws0_ref, wn0_ref, g0_ref, b0_ref,
              ws1_ref, wn1_ref, g1_ref, b1_ref, out_hbm,
              xbuf, obuf, y0_ref, y1_ref, insem, outsem):
    gh, gw = _GH, _GW
    vi = jax.lax.broadcasted_iota(jnp.int32, (v, d), 0)
    gi = vi // gw
    gj = vi % gw
    deg = ((gi > 0).astype(jnp.float32) + (gi < gh - 1).astype(jnp.float32)
           + (gj > 0).astype(jnp.float32) + (gj < gw - 1).astype(jnp.float32))
    dinv = jax.lax.rsqrt(deg)
    dinv3 = dinv.reshape(gh, gw, d)
    zi = jnp.zeros((1, gw, d), dtype=jnp.float32)
    zj = jnp.zeros((gh, 1, d), dtype=jnp.float32)
    zero_row = jnp.zeros((1, d), dtype=jnp.float32)
    cnt = jnp.float32(n * v)
    dims = (((1,), (1,)), ((), ()))

    def layer_mm(x, ws_b, wn_b):
        xs = x.reshape(gh, gw, d) * dinv3
        u = (jnp.concatenate([zi, xs[:-1]], axis=0)
             + jnp.concatenate([xs[1:], zi], axis=0)
             + jnp.concatenate([zj, xs[:, :-1]], axis=1)
             + jnp.concatenate([xs[:, 1:], zj], axis=1))
        s = jax.lax.dot_general(x.astype(jnp.bfloat16), ws_b, dims,
                                preferred_element_type=jnp.float32)
        r = jax.lax.dot_general(u.reshape(v, d).astype(jnp.bfloat16), wn_b,
                                dims, preferred_element_type=jnp.float32)
        return s + dinv * r

    def bn_consts(s_acc, q_acc, g_ref, b_ref):
        mu = s_acc / cnt
        var = q_acc / cnt - mu * mu
        scale = jax.lax.rsqrt(var + 1e-5) * g_ref[...]
        return scale, b_ref[...] - mu * scale

    ws0_b = ws0_ref[...].astype(jnp.bfloat16)
    wn0_b = wn0_ref[...].astype(jnp.bfloat16)
    ws1_b = ws1_ref[...].astype(jnp.bfloat16)
    wn1_b = wn1_ref[...].astype(jnp.bfloat16)

    in_copies = [
        pltpu.make_async_copy(h_hbm.at[b], xbuf.at[b % 2], insem.at[b % 2])
        for b in range(n)
    ]
    in_copies[0].start()
    s0 = q0 = zero_row
    for b in range(n):
        if b + 1 < n:
            in_copies[b + 1].start()
        in_copies[b].wait()
        y = layer_mm(xbuf[b % 2], ws0_b, wn0_b)
        y0_ref[pl.ds(b * v, v), :] = y
        s0 = s0 + jnp.sum(y, axis=0, keepdims=True)
        q0 = q0 + jnp.sum(y * y, axis=0, keepdims=True)

    scale0, off0 = bn_consts(s0, q0, g0_ref, b0_ref)
    s1 = q1 = zero_row
    for b in range(n):
        x1 = jnp.maximum(y0_ref[pl.ds(b * v, v), :] * scale0 + off0, 0.0)
        y = layer_mm(x1, ws1_b, wn1_b)
        y1_ref[pl.ds(b * v, v), :] = y
        s1 = s1 + jnp.sum(y, axis=0, keepdims=True)
        q1 = q1 + jnp.sum(y * y, axis=0, keepdims=True)

    scale1, off1 = bn_consts(s1, q1, g1_ref, b1_ref)
    out_copies = [
        pltpu.make_async_copy(obuf.at[b % 2], out_hbm.at[b], outsem.at[b % 2])
        for b in range(n)
    ]
    for b in range(n):
        if b >= 2:
            out_copies[b - 2].wait()
        obuf[b % 2] = jnp.maximum(
            y1_ref[pl.ds(b * v, v), :] * scale1 + off1, 0.0)
        out_copies[b].start()
    out_copies[n - 2].wait()
    out_copies[n - 1].wait()


def kernel(H, A_norm, Ws0, Wn0, g0, b0, Ws1, Wn1, g1, b1):
    n, v, d = H.shape
    body = functools.partial(_gnn_body, n, v, d)
    return pl.pallas_call(
        body,
        out_shape=jax.ShapeDtypeStruct((n, v, d), jnp.float32),
        in_specs=[pl.BlockSpec(memory_space=pl.ANY)]
        + [pl.BlockSpec(memory_space=pltpu.MemorySpace.VMEM)] * 8,
        out_specs=pl.BlockSpec(memory_space=pl.ANY),
        scratch_shapes=[
            pltpu.VMEM((2, v, d), jnp.float32),
            pltpu.VMEM((2, v, d), jnp.float32),
            pltpu.VMEM((n * v, d), jnp.float32),
            pltpu.VMEM((n * v, d), jnp.float32),
            pltpu.SemaphoreType.DMA((2,)),
            pltpu.SemaphoreType.DMA((2,)),
        ],
    )(H, Ws0, Wn0, g0.reshape(1, d), b0.reshape(1, d),
      Ws1, Wn1, g1.reshape(1, d), b1.reshape(1, d))
